# baseline (speedup 1.0000x reference)
import functools

import jax
import jax.numpy as jnp
from jax import lax
from jax.experimental import pallas as pl
from jax.experimental.pallas import tpu as pltpu
from jax.experimental.pallas import tpu_sc as plsc

EMBED = 64
LANE = 128
CHUNK = 100
NBUF = 6
LOOK = 2


def _table_pad_body(t_ref, o_ref):
    blk = t_ref[...]
    o_ref[:, :EMBED] = jnp.swapaxes(blk, 0, 1)
    o_ref[:, EMBED:] = jnp.zeros_like(o_ref[:, EMBED:])


def _out_t_body(y_ref, o_ref):
    blk = y_ref[...]
    for tt in range(blk.shape[1]):
        o_ref[tt] = jnp.swapaxes(blk[:, tt, :EMBED], 0, 1)


def _sc_body(x_hbm, table_hbm, pos_hbm, out_hbm, idx_v, pos_sh, bufs,
             psem, gsem, osem, *, num_cores, chunks, per_w):
    sid = lax.axis_index("s")
    wid = sid * num_cores + lax.axis_index("c")
    pltpu.sync_copy(x_hbm.at[wid], idx_v)

    @pl.when(sid == 0)
    def _():
        for h in range(2):
            sl = pl.ds(h * CHUNK, CHUNK)
            pltpu.sync_copy(pos_hbm.at[sl], bufs.at[0])
            pltpu.sync_copy(bufs.at[0], pos_sh.at[sl])

    plsc.subcore_barrier()

    def step(i, carry):
        b = lax.rem(i, NBUF)
        @pl.when(i >= NBUF)
        def _():
            pltpu.make_async_copy(
                bufs.at[b], out_hbm.at[pl.ds(0, CHUNK)], osem.at[b]).wait()

        @pl.when(i < chunks)
        def _():
            pbase = lax.rem(i, 2) * CHUNK
            pltpu.async_copy(
                pos_sh.at[pl.ds(pbase, CHUNK)], bufs.at[b], psem.at[b])

        @pl.when(jnp.logical_and(i >= LOOK, i < chunks + LOOK))
        def _():
            j = i - LOOK
            bj = lax.rem(j, NBUF)
            pltpu.make_async_copy(
                pos_sh.at[pl.ds(0, CHUNK)], bufs.at[bj], psem.at[bj]).wait()
            pltpu.async_copy(
                table_hbm.at[idx_v.at[j]], bufs.at[bj], gsem.at[bj],
                add=True)

        @pl.when(jnp.logical_and(i >= 2 * LOOK, i < chunks + 2 * LOOK))
        def _():
            j = i - 2 * LOOK
            bj = lax.rem(j, NBUF)
            pltpu.make_async_copy(
                table_hbm.at[idx_v.at[j]], bufs.at[bj], gsem.at[bj]).wait()
            row0 = wid * per_w + j * CHUNK
            pltpu.async_copy(
                bufs.at[bj], out_hbm.at[pl.ds(row0, CHUNK)], osem.at[bj])

        return carry

    lax.fori_loop(0, chunks + NBUF, step, 0, unroll=False)


def kernel(x, token_emb, pos_emb):
    B, T = x.shape
    V = token_emb.shape[0]
    info = plsc.get_sparse_core_info()
    nw = info.num_cores * info.num_subcores
    total = B * T
    per_w = total // nw
    chunks = per_w // CHUNK
    assert per_w % CHUNK == 0 and per_w % T == 0 and T == 2 * CHUNK
    assert chunks % 2 == 0 and chunks >= NBUF

    vblocks = pl.cdiv(V, LANE)
    t128 = pl.pallas_call(
        _table_pad_body,
        grid=(vblocks,),
        in_specs=[pl.BlockSpec((EMBED, LANE), lambda j: (0, j))],
        out_specs=pl.BlockSpec((LANE, LANE), lambda j: (j, 0)),
        out_shape=jax.ShapeDtypeStruct((V, LANE), jnp.float32),
    )(token_emb.T)

    x_r = x.astype(jnp.int32).reshape(nw, chunks, CHUNK)
    pos128 = jnp.pad(pos_emb[0, :T, :], ((0, 0), (0, LANE - EMBED)))

    mesh = plsc.VectorSubcoreMesh(core_axis_name="c", subcore_axis_name="s")
    body = functools.partial(_sc_body, num_cores=info.num_cores,
                             chunks=chunks, per_w=per_w)
    y128 = pl.kernel(
        body,
        out_type=jax.ShapeDtypeStruct((total, LANE), jnp.float32),
        mesh=mesh,
        compiler_params=pltpu.CompilerParams(use_tc_tiling_on_sc=False),
        scratch_types=[
            pltpu.VMEM((chunks, CHUNK), jnp.int32),
            pltpu.VMEM_SHARED((T, LANE), jnp.float32),
            pltpu.VMEM((NBUF, CHUNK, LANE), jnp.float32),
            pltpu.SemaphoreType.DMA((NBUF,)),
            pltpu.SemaphoreType.DMA((NBUF,)),
            pltpu.SemaphoreType.DMA((NBUF,)),
        ],
    )(x_r, t128, pos128)

    y3 = y128.reshape(B, T, LANE)
    yt = pl.pallas_call(
        _out_t_body,
        grid=(B // LANE, T // 8),
        in_specs=[pl.BlockSpec((LANE, 8, LANE), lambda jb, tb: (jb, tb, 0))],
        out_specs=pl.BlockSpec((8, EMBED, LANE), lambda jb, tb: (tb, 0, jb)),
        out_shape=jax.ShapeDtypeStruct((T, EMBED, B), jnp.float32),
    )(y3)
    return jnp.transpose(yt, (2, 0, 1))


# speedup vs baseline: 4.9231x; 4.9231x over previous
import functools

import jax
import jax.numpy as jnp
from jax import lax
from jax.experimental import pallas as pl
from jax.experimental.pallas import tpu as pltpu
from jax.experimental.pallas import tpu_sc as plsc

EMBED = 64
LANE = 128
CHUNK = 100
NBUF = 6
LOOK = 2


def _table_pad_body(t_ref, o_ref):
    blk = t_ref[...]
    o_ref[:, :EMBED] = jnp.swapaxes(blk, 0, 1)
    o_ref[:, EMBED:] = jnp.zeros_like(o_ref[:, EMBED:])


def _out_t_body(y_ref, o_ref):
    blk = y_ref[...]
    for tt in range(blk.shape[1]):
        o_ref[tt] = jnp.swapaxes(blk[:, tt, :EMBED], 0, 1)


def _sc_body(x_hbm, table_hbm, pos_hbm, out_hbm, idx_v, pos_sh, bufs,
             psem, gsem, osem, *, num_cores, chunks, per_w):
    sid = lax.axis_index("s")
    wid = sid * num_cores + lax.axis_index("c")
    pltpu.sync_copy(x_hbm.at[wid], idx_v)

    @pl.when(sid == 0)
    def _():
        for h in range(2):
            sl = pl.ds(h * CHUNK, CHUNK)
            pltpu.sync_copy(pos_hbm.at[sl], bufs.at[0])
            pltpu.sync_copy(bufs.at[0], pos_sh.at[sl])

    plsc.subcore_barrier()

    def step(i, carry):
        b = lax.rem(i, NBUF)
        @pl.when(i >= NBUF)
        def _():
            pltpu.make_async_copy(
                bufs.at[b], out_hbm.at[pl.ds(0, CHUNK)], osem.at[b]).wait()

        @pl.when(i < chunks)
        def _():
            pbase = lax.rem(i, 2) * CHUNK
            pltpu.async_copy(
                pos_sh.at[pl.ds(pbase, CHUNK)], bufs.at[b], psem.at[b])

        @pl.when(jnp.logical_and(i >= LOOK, i < chunks + LOOK))
        def _():
            j = i - LOOK
            bj = lax.rem(j, NBUF)
            pltpu.make_async_copy(
                pos_sh.at[pl.ds(0, CHUNK)], bufs.at[bj], psem.at[bj]).wait()
            pltpu.async_copy(
                table_hbm.at[idx_v.at[j]], bufs.at[bj], gsem.at[bj],
                add=True)

        @pl.when(jnp.logical_and(i >= 2 * LOOK, i < chunks + 2 * LOOK))
        def _():
            j = i - 2 * LOOK
            bj = lax.rem(j, NBUF)
            pltpu.make_async_copy(
                table_hbm.at[idx_v.at[j]], bufs.at[bj], gsem.at[bj]).wait()
            row0 = wid * per_w + j * CHUNK
            pltpu.async_copy(
                bufs.at[bj], out_hbm.at[pl.ds(row0, CHUNK)], osem.at[bj])

        return carry

    lax.fori_loop(0, chunks + NBUF, step, 0, unroll=False)


def kernel(x, token_emb, pos_emb):
    B, T = x.shape
    V = token_emb.shape[0]
    info = plsc.get_sparse_core_info()
    nw = info.num_cores * info.num_subcores
    total = B * T
    per_w = total // nw
    chunks = per_w // CHUNK
    assert per_w % CHUNK == 0 and per_w % T == 0 and T == 2 * CHUNK
    assert chunks % 2 == 0 and chunks >= NBUF

    t128 = jnp.pad(token_emb, ((0, 0), (0, LANE - EMBED)))

    x_r = x.astype(jnp.int32).reshape(nw, chunks, CHUNK)
    pos128 = jnp.pad(pos_emb[0, :T, :], ((0, 0), (0, LANE - EMBED)))

    mesh = plsc.VectorSubcoreMesh(core_axis_name="c", subcore_axis_name="s")
    body = functools.partial(_sc_body, num_cores=info.num_cores,
                             chunks=chunks, per_w=per_w)
    y128 = pl.kernel(
        body,
        out_type=jax.ShapeDtypeStruct((total, LANE), jnp.float32),
        mesh=mesh,
        compiler_params=pltpu.CompilerParams(use_tc_tiling_on_sc=False),
        scratch_types=[
            pltpu.VMEM((chunks, CHUNK), jnp.int32),
            pltpu.VMEM_SHARED((T, LANE), jnp.float32),
            pltpu.VMEM((NBUF, CHUNK, LANE), jnp.float32),
            pltpu.SemaphoreType.DMA((NBUF,)),
            pltpu.SemaphoreType.DMA((NBUF,)),
            pltpu.SemaphoreType.DMA((NBUF,)),
        ],
    )(x_r, t128, pos128)

    return y128.reshape(B, T, LANE)[:, :, :EMBED]


# speedup vs baseline: 5.7718x; 1.1724x over previous
import functools

import jax
import jax.numpy as jnp
from jax import lax
from jax.experimental import pallas as pl
from jax.experimental.pallas import tpu as pltpu
from jax.experimental.pallas import tpu_sc as plsc

EMBED = 64
LANE = 128
CHUNK = 100
NBUF = 6
LOOK = 2


def _table_pad_body(t_ref, o_ref):
    blk = t_ref[...]
    o_ref[:, :EMBED] = jnp.swapaxes(blk, 0, 1)
    o_ref[:, EMBED:] = jnp.zeros_like(o_ref[:, EMBED:])


def _out_t_body(y_ref, o_ref):
    blk = y_ref[...]
    for tt in range(blk.shape[1]):
        o_ref[tt] = jnp.swapaxes(blk[:, tt, :EMBED], 0, 1)


def _sc_body(x_hbm, table_hbm, pos_hbm, out_hbm, idx_v, pos_sh, bufs,
             psem, gsem, osem, *, num_cores, chunks, per_w):
    sid = lax.axis_index("s")
    wid = sid * num_cores + lax.axis_index("c")
    pltpu.sync_copy(x_hbm.at[wid], idx_v)

    @pl.when(sid == 0)
    def _():
        for h in range(2):
            sl = pl.ds(h * CHUNK, CHUNK)
            pltpu.sync_copy(pos_hbm.at[sl], bufs.at[0])
            pltpu.sync_copy(bufs.at[0], pos_sh.at[sl])

    plsc.subcore_barrier()

    def step(i, carry):
        b = lax.rem(i, NBUF)
        @pl.when(i >= NBUF)
        def _():
            pltpu.make_async_copy(
                bufs.at[b],
                out_hbm.at[pl.ds(0, CHUNK), pl.ds(0, EMBED)],
                osem.at[b]).wait()

        @pl.when(i < chunks)
        def _():
            pbase = lax.rem(i, 2) * CHUNK
            pltpu.async_copy(
                pos_sh.at[pl.ds(pbase, CHUNK)], bufs.at[b], psem.at[b])

        @pl.when(jnp.logical_and(i >= LOOK, i < chunks + LOOK))
        def _():
            j = i - LOOK
            bj = lax.rem(j, NBUF)
            pltpu.make_async_copy(
                pos_sh.at[pl.ds(0, CHUNK)], bufs.at[bj], psem.at[bj]).wait()
            pltpu.async_copy(
                table_hbm.at[idx_v.at[j]], bufs.at[bj], gsem.at[bj],
                add=True)

        @pl.when(jnp.logical_and(i >= 2 * LOOK, i < chunks + 2 * LOOK))
        def _():
            j = i - 2 * LOOK
            bj = lax.rem(j, NBUF)
            pltpu.make_async_copy(
                table_hbm.at[idx_v.at[j]], bufs.at[bj], gsem.at[bj]).wait()
            row0 = wid * per_w + j * CHUNK
            pltpu.async_copy(
                bufs.at[bj],
                out_hbm.at[pl.ds(row0, CHUNK), pl.ds(0, EMBED)],
                osem.at[bj])

        return carry

    lax.fori_loop(0, chunks + NBUF, step, 0, unroll=False)


def kernel(x, token_emb, pos_emb):
    B, T = x.shape
    V = token_emb.shape[0]
    info = plsc.get_sparse_core_info()
    nw = info.num_cores * info.num_subcores
    total = B * T
    per_w = total // nw
    chunks = per_w // CHUNK
    assert per_w % CHUNK == 0 and per_w % T == 0 and T == 2 * CHUNK
    assert chunks % 2 == 0 and chunks >= NBUF

    t128 = jnp.pad(token_emb, ((0, 0), (0, LANE - EMBED)))
    t64 = t128.reshape(2 * V, EMBED)

    x_r = (x.astype(jnp.int32) * 2).reshape(nw, chunks, CHUNK)
    pos2d = pos_emb[0, :T, :]

    mesh = plsc.VectorSubcoreMesh(core_axis_name="c", subcore_axis_name="s")
    body = functools.partial(_sc_body, num_cores=info.num_cores,
                             chunks=chunks, per_w=per_w)
    y128 = pl.kernel(
        body,
        out_type=jax.ShapeDtypeStruct((total, LANE), jnp.float32),
        mesh=mesh,
        compiler_params=pltpu.CompilerParams(use_tc_tiling_on_sc=False),
        scratch_types=[
            pltpu.VMEM((chunks, CHUNK), jnp.int32),
            pltpu.VMEM_SHARED((T, EMBED), jnp.float32),
            pltpu.VMEM((NBUF, CHUNK, EMBED), jnp.float32),
            pltpu.SemaphoreType.DMA((NBUF,)),
            pltpu.SemaphoreType.DMA((NBUF,)),
            pltpu.SemaphoreType.DMA((NBUF,)),
        ],
    )(x_r, t64, pos2d)

    return y128.reshape(B, T, LANE)[:, :, :EMBED]
